# baseline (speedup 1.0000x reference)
import functools

import jax
import jax.numpy as jnp
from jax.experimental import pallas as pl
from jax.experimental.pallas import tpu as pltpu
from jax.experimental.pallas import tpu_sc as plsc

N = 2048
H = 512
NH = 8
DH = H // NH
K = 1024
RT = 256
QT = 512

_SC_NC = 2
_SC_NS = 16
_NW = _SC_NC * _SC_NS
_BPW = N // _NW


def _lrelu(x):
    return jnp.where(x >= 0, x, 0.01 * x)


def _dotT(a, b):
    return jax.lax.dot_general(a, b, (((1,), (1,)), ((), ())),
                               preferred_element_type=jnp.float32)


def _enc_body(fch_ref, inp_ref, wfe_ref, bfe_ref, win_ref, bin_ref,
              whf_ref, whi_ref, bh_ref, wq_ref, wk_ref, wv_ref,
              fc_ref, q_ref, k_ref, v_ref):
    fc = _lrelu(jnp.dot(fch_ref[...], wfe_ref[...],
                        preferred_element_type=jnp.float32) + bfe_ref[...])
    t = jnp.dot(inp_ref[...], win_ref[...],
                preferred_element_type=jnp.float32) + bin_ref[...]
    x = _lrelu(jnp.dot(fc, whf_ref[...], preferred_element_type=jnp.float32)
               + jnp.dot(t, whi_ref[...], preferred_element_type=jnp.float32)
               + bh_ref[...])
    fc_ref[...] = fc
    q = jnp.dot(x, wq_ref[...], preferred_element_type=jnp.float32)
    k = jnp.dot(x, wk_ref[...], preferred_element_type=jnp.float32)
    v = jnp.dot(x, wv_ref[...], preferred_element_type=jnp.float32)
    for h in range(NH):
        q_ref[h] = q[:, h * DH:(h + 1) * DH]
        k_ref[h] = k[:, h * DH:(h + 1) * DH]
        v_ref[h] = v[:, h * DH:(h + 1) * DH]


def _attn_body(q_ref, k_ref, v_ref, o_ref):
    s = _dotT(q_ref[0], k_ref[0]) * 0.125
    m = jnp.max(s, axis=1, keepdims=True)
    e = jnp.exp(s - m)
    p = e / jnp.sum(e, axis=1, keepdims=True)
    o_ref[0] = jnp.dot(p, v_ref[0], preferred_element_type=jnp.float32)


def _vq_body(attn_ref, wo_ref, bo_ref, cb_ref, ze_ref, idx_ref):
    a = jnp.concatenate([attn_ref[h] for h in range(NH)], axis=1)
    z = jnp.dot(a, wo_ref[...],
                preferred_element_type=jnp.float32) + bo_ref[...]
    cb = cb_ref[...]
    cn = jnp.sum(cb * cb, axis=1)[None, :]
    zn = jnp.sum(z * z, axis=1, keepdims=True)
    d = zn - 2.0 * _dotT(z, cb) + cn
    dmin = jnp.min(d, axis=1, keepdims=True)
    ii = jax.lax.broadcasted_iota(jnp.int32, d.shape, 1)
    idx = jnp.min(jnp.where(d == dmin, ii, K), axis=1)
    ze_ref[...] = z
    idx_ref[0, 0, :] = idx


def _dec_body(fc_ref, ze_ref, zq_ref, wda_ref, wdb_ref, bd1_ref,
              wd2_ref, bd2_ref, out_ref, zst_ref):
    ze = ze_ref[...]
    zst = ze + (zq_ref[...] - ze)
    h = _lrelu(jnp.dot(fc_ref[...], wda_ref[...],
                       preferred_element_type=jnp.float32)
               + jnp.dot(zst, wdb_ref[...],
                         preferred_element_type=jnp.float32)
               + bd1_ref[...])
    out_ref[...] = jnp.dot(h, wd2_ref[...],
                           preferred_element_type=jnp.float32) + bd2_ref[...]
    zst_ref[...] = zst


def _sc_gather(codebook, idx):
    mesh = plsc.VectorSubcoreMesh(core_axis_name="c", subcore_axis_name="s")

    @functools.partial(
        pl.kernel, mesh=mesh,
        out_type=jax.ShapeDtypeStruct((N, H), jnp.float32),
        scratch_types=[
            pltpu.VMEM((_BPW,), jnp.int32),
            pltpu.VMEM((_BPW, H), jnp.float32),
            pltpu.SemaphoreType.DMA,
        ],
    )
    def gk(cb_hbm, idx_hbm, out_hbm, idx_v, rows_v, sem):
        wid = jax.lax.axis_index("s") * _SC_NC + jax.lax.axis_index("c")
        base = wid * _BPW
        pltpu.sync_copy(idx_hbm.at[pl.ds(base, _BPW)], idx_v)
        pltpu.async_copy(cb_hbm.at[idx_v], rows_v, sem).wait()
        pltpu.sync_copy(rows_v, out_hbm.at[pl.ds(base, _BPW)])

    return gk(codebook, idx)


def kernel(input, firm_char, W_fe, b_fe, W_in, b_in, W_h, b_h,
           W_q, W_k, W_v, W_o, b_o, codebook, W_d1, b_d1, W_d2, b_d2):
    f32 = jnp.float32
    nrt = N // RT

    full = lambda a: pl.BlockSpec(a.shape, lambda i: (0,) * a.ndim)
    row = lambda c: pl.BlockSpec((RT, c), lambda i: (i, 0))

    fc, q, k, v = pl.pallas_call(
        _enc_body,
        grid=(nrt,),
        in_specs=[row(256), row(64),
                  full(W_fe), pl.BlockSpec((1, H), lambda i: (0, 0)),
                  full(W_in), pl.BlockSpec((1, H), lambda i: (0, 0)),
                  pl.BlockSpec((H, H), lambda i: (0, 0)),
                  pl.BlockSpec((H, H), lambda i: (0, 0)),
                  pl.BlockSpec((1, H), lambda i: (0, 0)),
                  full(W_q), full(W_k), full(W_v)],
        out_specs=[row(H),
                   pl.BlockSpec((NH, RT, DH), lambda i: (0, i, 0)),
                   pl.BlockSpec((NH, RT, DH), lambda i: (0, i, 0)),
                   pl.BlockSpec((NH, RT, DH), lambda i: (0, i, 0))],
        out_shape=[jax.ShapeDtypeStruct((N, H), f32)]
                  + [jax.ShapeDtypeStruct((NH, N, DH), f32)] * 3,
    )(firm_char, input, W_fe, b_fe.reshape(1, H), W_in, b_in.reshape(1, H),
      W_h[:H], W_h[H:], b_h.reshape(1, H), W_q, W_k, W_v)

    attn = pl.pallas_call(
        _attn_body,
        grid=(NH, N // QT),
        in_specs=[pl.BlockSpec((1, QT, DH), lambda h, i: (h, i, 0)),
                  pl.BlockSpec((1, N, DH), lambda h, i: (h, 0, 0)),
                  pl.BlockSpec((1, N, DH), lambda h, i: (h, 0, 0))],
        out_specs=pl.BlockSpec((1, QT, DH), lambda h, i: (h, i, 0)),
        out_shape=jax.ShapeDtypeStruct((NH, N, DH), f32),
    )(q, k, v)

    z_e, idx3 = pl.pallas_call(
        _vq_body,
        grid=(nrt,),
        in_specs=[pl.BlockSpec((NH, RT, DH), lambda i: (0, i, 0)),
                  full(W_o), pl.BlockSpec((1, H), lambda i: (0, 0)),
                  full(codebook)],
        out_specs=[row(H), pl.BlockSpec((1, 1, RT), lambda i: (i, 0, 0))],
        out_shape=[jax.ShapeDtypeStruct((N, H), f32),
                   jax.ShapeDtypeStruct((nrt, 1, RT), jnp.int32)],
    )(attn, W_o, b_o.reshape(1, H), codebook)

    idx = idx3.reshape(N)
    z_q_raw = _sc_gather(codebook, idx)

    output, z_q = pl.pallas_call(
        _dec_body,
        grid=(nrt,),
        in_specs=[row(H), row(H), row(H),
                  pl.BlockSpec((H, H), lambda i: (0, 0)),
                  pl.BlockSpec((H, H), lambda i: (0, 0)),
                  pl.BlockSpec((1, H), lambda i: (0, 0)),
                  full(W_d2), pl.BlockSpec((1, 1), lambda i: (0, 0))],
        out_specs=[pl.BlockSpec((RT, 1), lambda i: (i, 0)), row(H)],
        out_shape=[jax.ShapeDtypeStruct((N, 1), f32),
                   jax.ShapeDtypeStruct((N, H), f32)],
    )(fc, z_e, z_q_raw, W_d1[:H], W_d1[H:], b_d1.reshape(1, H),
      W_d2, b_d2.reshape(1, 1))

    return output, z_q, idx
